# baseline (speedup 1.0000x reference)
import jax
import jax.numpy as jnp
from jax import lax
from jax.experimental import pallas as pl
from jax.experimental.pallas import tpu as pltpu
from jax.experimental.pallas import tpu_sc as plsc

_SEQ = 4096
_HID = 4096
_SLOTS = 64
_K = 8
_LANES = 16
_NTILES = 16
_TC_ROWS = 2048
_SC_ROWS = _SEQ - _TC_ROWS
_RPT = _SC_ROWS // 32
_CHUNK = 8
_NCHUNKS = _RPT // _CHUNK
_PER_TILE = _SEQ // _NTILES
_NCHUNK_TOPK = _PER_TILE // _LANES


def _norms_body(x_ref, o_ref):
    x = x_ref[...]
    o_ref[...] = jnp.sum(x * x, axis=1)[None, None, :]


def _tc_norms(pred2d):
    nblk = 4
    rows = _TC_ROWS // nblk
    return pl.pallas_call(
        _norms_body,
        grid=(nblk,),
        in_specs=[pl.BlockSpec((rows, _HID), lambda i: (i, 0))],
        out_specs=pl.BlockSpec((1, 1, rows), lambda i: (i, 0, 0)),
        out_shape=jax.ShapeDtypeStruct((nblk, 1, rows), jnp.float32),
    )(pred2d)


def _sc_norms_body(pred_hbm, out_hbm, buf_a, buf_b, norms64, sem_a, sem_b):
    c = lax.axis_index("c")
    s = lax.axis_index("s")
    tile = c * _NTILES + s
    row0 = _TC_ROWS + tile * _RPT
    lane = lax.iota(jnp.int32, _LANES)

    bufs = (buf_a, buf_b)
    sems = (sem_a, sem_b)

    def chunk_src(ci):
        return pred_hbm.at[pl.ds(row0 + ci * _CHUNK, _CHUNK)]

    pltpu.make_async_copy(chunk_src(0), buf_a, sem_a).start()
    pltpu.make_async_copy(chunk_src(1), buf_b, sem_b).start()

    acc_vec = jnp.zeros((_LANES,), jnp.float32)
    for ci in range(_NCHUNKS):
        buf = bufs[ci % 2]
        sem = sems[ci % 2]
        pltpu.make_async_copy(chunk_src(ci), buf, sem).wait()
        for r in range(_CHUNK):
            def body16(j, acc):
                base = j * (_LANES * _LANES)
                for t in range(_LANES):
                    x = buf[r, pl.ds(base + t * _LANES, _LANES)]
                    acc = acc + x * x
                return acc

            acc = lax.fori_loop(0, _HID // (_LANES * _LANES), body16,
                                jnp.zeros((_LANES,), jnp.float32))
            total = lax.reduce_sum_p.bind(acc, axes=(0,))
            row_g = ci * _CHUNK + r
            acc_vec = jnp.where(lane == (row_g % _LANES), total, acc_vec)
            if row_g % _LANES == _LANES - 1:
                norms64[pl.ds((row_g // _LANES) * _LANES, _LANES)] = acc_vec
                acc_vec = jnp.zeros((_LANES,), jnp.float32)
        if ci + 2 < _NCHUNKS:
            pltpu.make_async_copy(chunk_src(ci + 2), buf, sem).start()
    pltpu.sync_copy(norms64, out_hbm.at[pl.ds(tile * _RPT, _RPT)])


def _sc_norms(pred2d):
    return pl.kernel(
        _sc_norms_body,
        mesh=plsc.VectorSubcoreMesh(core_axis_name="c", subcore_axis_name="s"),
        compiler_params=pltpu.CompilerParams(needs_layout_passes=False),
        out_type=jax.ShapeDtypeStruct((_SC_ROWS,), jnp.float32),
        scratch_types=[
            pltpu.VMEM((_CHUNK, _HID), jnp.float32),
            pltpu.VMEM((_CHUNK, _HID), jnp.float32),
            pltpu.VMEM((_RPT,), jnp.float32),
            pltpu.SemaphoreType.DMA,
            pltpu.SemaphoreType.DMA,
        ],
    )(pred2d)


def _merge_sorted(bk, bi, ck_s, ci_s):
    ck_r = lax.rev(ck_s, (0,))
    ci_r = lax.rev(ci_s, (0,))
    keep = bk >= ck_r
    mk = jnp.where(keep, bk, ck_r)
    mi = jnp.where(keep, bi, ci_r)
    nk, ni = plsc.sort_key_val(mk, mi, descending=True)
    return nk, ni


def _sc_topk_body(norms_tc_hbm, norms_sc_hbm, pred_hbm, states_hbm,
                  strength_hbm, out_states_hbm, out_strength_hbm,
                  norms_v, kv, iv, kvf, ivf, idx_v, rows_v, str_v, bank_v,
                  sh_k, sh_i, sem):
    c = lax.axis_index("c")
    s = lax.axis_index("s")

    @pl.when(c == 0)
    def _topk():
        base = pl.multiple_of(s * _PER_TILE, _PER_TILE)

        @pl.when(s < _NTILES // 2)
        def _load_tc():
            pltpu.sync_copy(norms_tc_hbm.at[pl.ds(base, _PER_TILE)], norms_v)

        @pl.when(s >= _NTILES // 2)
        def _load_sc():
            base2 = pl.multiple_of((s - _NTILES // 2) * _PER_TILE, _PER_TILE)
            pltpu.sync_copy(norms_sc_hbm.at[pl.ds(base2, _PER_TILE)], norms_v)

        lane = lax.iota(jnp.int32, _LANES)

        def local_merge(j, carry):
            bk, bi = carry
            ck = norms_v[pl.ds(j * _LANES, _LANES)]
            ci = lane + (base + j * _LANES)
            ck_s, ci_s = plsc.sort_key_val(ck, ci, descending=True)
            return _merge_sorted(bk, bi, ck_s, ci_s)

        bk0 = jnp.full((_LANES,), -jnp.inf, jnp.float32)
        bi0 = jnp.zeros((_LANES,), jnp.int32)
        bk, bi = lax.fori_loop(0, _NCHUNK_TOPK, local_merge, (bk0, bi0))
        kv[...] = bk
        iv[...] = bi
        pltpu.sync_copy(kv, sh_k.at[pl.ds(s * _LANES, _LANES)])
        pltpu.sync_copy(iv, sh_i.at[pl.ds(s * _LANES, _LANES)])
        plsc.subcore_barrier()

        @pl.when(s == 0)
        def _final():
            pltpu.sync_copy(sh_k, kvf)
            pltpu.sync_copy(sh_i, ivf)

            def final_merge(j, carry):
                bk2, bi2 = carry
                ck_s = kvf[pl.ds(j * _LANES, _LANES)]
                ci_s = ivf[pl.ds(j * _LANES, _LANES)]
                return _merge_sorted(bk2, bi2, ck_s, ci_s)

            fk, fi = lax.fori_loop(0, _NTILES, final_merge, (bk0, bi0))
            idx_v[...] = fi
            pltpu.async_copy(pred_hbm.at[idx_v], rows_v, sem).wait()
            pltpu.sync_copy(rows_v.at[pl.ds(0, _K)],
                            out_states_hbm.at[pl.ds(0, _K)])
            pltpu.sync_copy(strength_hbm, str_v)
            s0 = str_v[pl.ds(0, _LANES)]
            str_v[pl.ds(0, _LANES)] = jnp.where(lane < _K, jnp.float32(1.0), s0)
            pltpu.sync_copy(str_v, out_strength_hbm)

    @pl.when((c == 1) & (s < 14))
    def _copy_bank():
        r0 = _K + s * 4
        pltpu.sync_copy(states_hbm.at[pl.ds(r0, 4)], bank_v)
        pltpu.sync_copy(bank_v, out_states_hbm.at[pl.ds(r0, 4)])


def _sc_topk(norms_tc, norms_sc, pred2d, mem_states, mem_strength):
    return pl.kernel(
        _sc_topk_body,
        mesh=plsc.VectorSubcoreMesh(core_axis_name="c", subcore_axis_name="s"),
        compiler_params=pltpu.CompilerParams(needs_layout_passes=False),
        out_type=[
            jax.ShapeDtypeStruct((_SLOTS, _HID), jnp.float32),
            jax.ShapeDtypeStruct((_SLOTS,), jnp.float32),
        ],
        scratch_types=[
            pltpu.VMEM((_PER_TILE,), jnp.float32),
            pltpu.VMEM((_LANES,), jnp.float32),
            pltpu.VMEM((_LANES,), jnp.int32),
            pltpu.VMEM((_NTILES * _LANES,), jnp.float32),
            pltpu.VMEM((_NTILES * _LANES,), jnp.int32),
            pltpu.VMEM((_LANES,), jnp.int32),
            pltpu.VMEM((_LANES, _HID), jnp.float32),
            pltpu.VMEM((_SLOTS,), jnp.float32),
            pltpu.VMEM((4, _HID), jnp.float32),
            pltpu.VMEM_SHARED((_NTILES * _LANES,), jnp.float32),
            pltpu.VMEM_SHARED((_NTILES * _LANES,), jnp.int32),
            pltpu.SemaphoreType.DMA,
        ],
    )(norms_tc, norms_sc, pred2d, mem_states, mem_strength)


def kernel(predictions, mem_states, mem_strength, top_k):
    del top_k
    pred2d = predictions.reshape(2 * _SEQ, _HID)
    norms_tc = _tc_norms(pred2d).reshape(_TC_ROWS)
    norms_sc = _sc_norms(pred2d)
    new_states, new_strength = _sc_topk(
        norms_tc, norms_sc, pred2d, mem_states, mem_strength)
    return new_states, new_strength


# speedup vs baseline: 1.4168x; 1.4168x over previous
import jax
import jax.numpy as jnp
from jax import lax
from jax.experimental import pallas as pl
from jax.experimental.pallas import tpu as pltpu
from jax.experimental.pallas import tpu_sc as plsc

_SEQ = 4096
_HID = 4096
_SLOTS = 64
_K = 8
_LANES = 16
_NTILES = 16
_PER_TILE = _SEQ // _NTILES
_NCHUNK = _PER_TILE // _LANES
_PACK = 2 * _LANES


def _norms_body(x_ref, o_ref):
    x = x_ref[...]
    o_ref[...] = jnp.sum(x * x, axis=1)[None, None, :]


def _tc_norms(pred2d):
    nblk = 8
    rows = _SEQ // nblk
    return pl.pallas_call(
        _norms_body,
        grid=(nblk,),
        in_specs=[pl.BlockSpec((rows, _HID), lambda i: (i, 0))],
        out_specs=pl.BlockSpec((1, 1, rows), lambda i: (i, 0, 0)),
        out_shape=jax.ShapeDtypeStruct((nblk, 1, rows), jnp.float32),
    )(pred2d)


def _merge_sorted(bk, bi, ck_s, ci_s):
    ck_r = lax.rev(ck_s, (0,))
    ci_r = lax.rev(ci_s, (0,))
    keep = bk >= ck_r
    mk = jnp.where(keep, bk, ck_r)
    mi = jnp.where(keep, bi, ci_r)
    nk, ni = plsc.sort_key_val(mk, mi, descending=True)
    return nk, ni


def _sc_body(norms_hbm, pred_hbm, states_hbm, strength_hbm,
             out_states_hbm, out_strength_hbm,
             norms_v, pack_v, packf, idx_v, rows_v, str_v, bank_v,
             sh_p, sem):
    c = lax.axis_index("c")
    s = lax.axis_index("s")

    @pl.when(c == 0)
    def _topk():
        base = pl.multiple_of(s * _PER_TILE, _PER_TILE)
        pltpu.sync_copy(norms_hbm.at[pl.ds(base, _PER_TILE)], norms_v)
        lane = lax.iota(jnp.int32, _LANES)

        def local_merge(j, carry):
            bk, bi = carry
            ck = norms_v[pl.ds(j * _LANES, _LANES)]
            ci = lane + (base + j * _LANES)
            ck_s, ci_s = plsc.sort_key_val(ck, ci, descending=True)
            return _merge_sorted(bk, bi, ck_s, ci_s)

        bk0 = jnp.full((_LANES,), -jnp.inf, jnp.float32)
        bi0 = jnp.zeros((_LANES,), jnp.int32)
        bk, bi = lax.fori_loop(0, _NCHUNK, local_merge, (bk0, bi0))
        pack_v[pl.ds(0, _LANES)] = bk
        pack_v[pl.ds(_LANES, _LANES)] = plsc.bitcast(bi, jnp.float32)
        pltpu.sync_copy(pack_v, sh_p.at[pl.ds(s * _PACK, _PACK)])
        plsc.subcore_barrier()

        @pl.when(s == 0)
        def _final():
            pltpu.sync_copy(sh_p, packf)

            def final_merge(j, carry):
                bk2, bi2 = carry
                ck_s = packf[pl.ds(j * _PACK, _LANES)]
                ci_s = plsc.bitcast(packf[pl.ds(j * _PACK + _LANES, _LANES)],
                                    jnp.int32)
                return _merge_sorted(bk2, bi2, ck_s, ci_s)

            fk, fi = lax.fori_loop(0, _NTILES, final_merge, (bk0, bi0))
            idx_v[...] = fi
            pltpu.async_copy(pred_hbm.at[idx_v.at[pl.ds(0, _K)]], rows_v,
                             sem).wait()
            pltpu.sync_copy(rows_v, out_states_hbm.at[pl.ds(0, _K)])
            pltpu.sync_copy(strength_hbm, str_v)
            s0 = str_v[pl.ds(0, _LANES)]
            str_v[pl.ds(0, _LANES)] = jnp.where(lane < _K, jnp.float32(1.0), s0)
            pltpu.sync_copy(str_v, out_strength_hbm)

    @pl.when((c == 1) & (s < 14))
    def _copy_bank():
        r0 = _K + s * 4
        pltpu.sync_copy(states_hbm.at[pl.ds(r0, 4)], bank_v)
        pltpu.sync_copy(bank_v, out_states_hbm.at[pl.ds(r0, 4)])


def kernel(predictions, mem_states, mem_strength, top_k):
    del top_k
    pred2d = predictions.reshape(2 * _SEQ, _HID)
    norms = _tc_norms(pred2d).reshape(_SEQ)
    sc = pl.kernel(
        _sc_body,
        mesh=plsc.VectorSubcoreMesh(core_axis_name="c", subcore_axis_name="s"),
        compiler_params=pltpu.CompilerParams(needs_layout_passes=False),
        out_type=[
            jax.ShapeDtypeStruct((_SLOTS, _HID), jnp.float32),
            jax.ShapeDtypeStruct((_SLOTS,), jnp.float32),
        ],
        scratch_types=[
            pltpu.VMEM((_PER_TILE,), jnp.float32),
            pltpu.VMEM((_PACK,), jnp.float32),
            pltpu.VMEM((_NTILES * _PACK,), jnp.float32),
            pltpu.VMEM((_LANES,), jnp.int32),
            pltpu.VMEM((_K, _HID), jnp.float32),
            pltpu.VMEM((_SLOTS,), jnp.float32),
            pltpu.VMEM((4, _HID), jnp.float32),
            pltpu.VMEM_SHARED((_NTILES * _PACK,), jnp.float32),
            pltpu.SemaphoreType.DMA,
        ],
    )
    new_states, new_strength = sc(norms, pred2d, mem_states, mem_strength)
    return new_states, new_strength


# speedup vs baseline: 1.4386x; 1.0154x over previous
import jax
import jax.numpy as jnp
from jax import lax
from jax.experimental import pallas as pl
from jax.experimental.pallas import tpu as pltpu
from jax.experimental.pallas import tpu_sc as plsc

_SEQ = 4096
_HID = 4096
_SLOTS = 64
_K = 8
_LANES = 16
_NTILES = 16
_PER_TILE = _SEQ // _NTILES
_NCHUNK = _PER_TILE // _LANES
_PACK = 2 * _LANES


def _norms_body(x_ref, o_ref):
    x = x_ref[...]
    o_ref[...] = jnp.sum(x * x, axis=1)[None, None, :]


def _tc_norms(pred2d):
    nblk = 8
    rows = _SEQ // nblk
    return pl.pallas_call(
        _norms_body,
        grid=(nblk,),
        in_specs=[pl.BlockSpec((rows, _HID), lambda i: (i, 0))],
        out_specs=pl.BlockSpec((1, 1, rows), lambda i: (i, 0, 0)),
        out_shape=jax.ShapeDtypeStruct((nblk, 1, rows), jnp.float32),
    )(pred2d)


def _merge_sorted(bk, bi, ck_s, ci_s):
    ck_r = lax.rev(ck_s, (0,))
    ci_r = lax.rev(ci_s, (0,))
    keep = bk >= ck_r
    mk = jnp.where(keep, bk, ck_r)
    mi = jnp.where(keep, bi, ci_r)
    nk, ni = plsc.sort_key_val(mk, mi, descending=True)
    return nk, ni


def _sc_body(norms_hbm, pred_hbm, states_hbm, strength_hbm,
             out_states_hbm, out_strength_hbm,
             norms_v, pack_v, packf, idx_v, rows_v, str_v, bank_v,
             sh_p, sem):
    c = lax.axis_index("c")
    s = lax.axis_index("s")

    @pl.when(c == 0)
    def _topk():
        base = pl.multiple_of(s * _PER_TILE, _PER_TILE)
        pltpu.sync_copy(norms_hbm.at[pl.ds(base, _PER_TILE)], norms_v)
        lane = lax.iota(jnp.int32, _LANES)

        def local_merge(j, carry):
            bk, bi = carry
            ck = norms_v[pl.ds(j * _LANES, _LANES)]
            ci = lane + (base + j * _LANES)
            ck_s, ci_s = plsc.sort_key_val(ck, ci, descending=True)
            return _merge_sorted(bk, bi, ck_s, ci_s)

        bk0 = jnp.full((_LANES,), -jnp.inf, jnp.float32)
        bi0 = jnp.zeros((_LANES,), jnp.int32)
        bk, bi = lax.fori_loop(0, _NCHUNK, local_merge, (bk0, bi0))
        pack_v[pl.ds(0, _LANES)] = bk
        pack_v[pl.ds(_LANES, _LANES)] = plsc.bitcast(bi, jnp.float32)
        pltpu.sync_copy(pack_v, sh_p.at[pl.ds(s * _PACK, _PACK)])
        plsc.subcore_barrier()

        @pl.when(s == 0)
        def _final():
            pltpu.sync_copy(sh_p, packf)

            def final_merge(j, carry):
                bk2, bi2 = carry
                ck_s = packf[pl.ds(j * _PACK, _LANES)]
                ci_s = plsc.bitcast(packf[pl.ds(j * _PACK + _LANES, _LANES)],
                                    jnp.int32)
                return _merge_sorted(bk2, bi2, ck_s, ci_s)

            fk, fi = lax.fori_loop(0, _NTILES, final_merge, (bk0, bi0))
            idx_v[...] = fi
            pltpu.async_copy(pred_hbm.at[idx_v.at[pl.ds(0, _K)]], rows_v,
                             sem).wait()
            pltpu.sync_copy(rows_v, out_states_hbm.at[pl.ds(0, _K)])

    @pl.when((c == 1) & (s < 14))
    def _copy_bank():
        r0 = _K + s * 4
        pltpu.sync_copy(states_hbm.at[pl.ds(r0, 4)], bank_v)
        pltpu.sync_copy(bank_v, out_states_hbm.at[pl.ds(r0, 4)])

    @pl.when((c == 1) & (s == 14))
    def _strength():
        lane = lax.iota(jnp.int32, _LANES)
        pltpu.sync_copy(strength_hbm, str_v)
        s0 = str_v[pl.ds(0, _LANES)]
        str_v[pl.ds(0, _LANES)] = jnp.where(lane < _K, jnp.float32(1.0), s0)
        pltpu.sync_copy(str_v, out_strength_hbm)


def kernel(predictions, mem_states, mem_strength, top_k):
    del top_k
    pred2d = predictions.reshape(2 * _SEQ, _HID)
    norms = _tc_norms(pred2d).reshape(_SEQ)
    sc = pl.kernel(
        _sc_body,
        mesh=plsc.VectorSubcoreMesh(core_axis_name="c", subcore_axis_name="s"),
        compiler_params=pltpu.CompilerParams(needs_layout_passes=False),
        out_type=[
            jax.ShapeDtypeStruct((_SLOTS, _HID), jnp.float32),
            jax.ShapeDtypeStruct((_SLOTS,), jnp.float32),
        ],
        scratch_types=[
            pltpu.VMEM((_PER_TILE,), jnp.float32),
            pltpu.VMEM((_PACK,), jnp.float32),
            pltpu.VMEM((_NTILES * _PACK,), jnp.float32),
            pltpu.VMEM((_LANES,), jnp.int32),
            pltpu.VMEM((_K, _HID), jnp.float32),
            pltpu.VMEM((_SLOTS,), jnp.float32),
            pltpu.VMEM((4, _HID), jnp.float32),
            pltpu.VMEM_SHARED((_NTILES * _PACK,), jnp.float32),
            pltpu.SemaphoreType.DMA,
        ],
    )
    new_states, new_strength = sc(norms, pred2d, mem_states, mem_strength)
    return new_states, new_strength
